# Initial kernel scaffold; baseline (speedup 1.0000x reference)
import jax
import jax.numpy as jnp
from jax.experimental import pallas as pl


def kernel(x, edge_index, Wr, br, Wz, bz, Wh, bh, Wg, bg):
    raise NotImplementedError("write your pallas kernel here")


# speedup vs baseline: 57.5255x; 57.5255x over previous
import functools

import jax
import jax.numpy as jnp
from jax import lax
from jax.experimental import pallas as pl
from jax.experimental.pallas import tpu as pltpu
from jax.experimental.pallas import tpu_sc as plsc

N = 10000
E = 160000
B = 4
D_IN = 256
H = 128

NPAD = 10240
DUMMY = 10016
NTILES = 16
CHUNK = 128
NCHUNK = 79
EPT = NCHUNK * CHUNK
NODES_PER_TILE = NPAD // NTILES
OUT_PER_TILE = NPAD // 2 // NTILES


def _rsqrt16(d):
    t = d
    for _ in range(14):
        t = 0.5 * (t + d / t)
    return 1.0 / t


def _sc_body(src_hbm, dst_hbm, out_hbm,
             src_v, dst_v, ones_v, vals_v, dbuf, nbuf, sbuf, ibuf, cbuf,
             deg_out_sh, deg_in_sh, s_sh):
    cid = lax.axis_index("c")
    sid = lax.axis_index("s")

    pltpu.sync_copy(src_hbm.at[sid], src_v)
    pltpu.sync_copy(dst_hbm.at[sid], dst_v)

    def _init_ones(j, carry):
        ones_v[pl.ds(j * 16, 16)] = jnp.full((16,), 1.0, jnp.float32)
        return carry
    lax.fori_loop(0, CHUNK // 16, _init_ones, 0)

    def _init_zero(j, carry):
        dbuf[pl.ds(j * 16, 16)] = jnp.zeros((16,), jnp.float32)
        return carry
    lax.fori_loop(0, NODES_PER_TILE // 16, _init_zero, 0)

    nb = sid * NODES_PER_TILE
    pltpu.sync_copy(dbuf, deg_out_sh.at[pl.ds(nb, NODES_PER_TILE)])
    pltpu.sync_copy(dbuf, deg_in_sh.at[pl.ds(nb, NODES_PER_TILE)])
    pltpu.sync_copy(dbuf, s_sh.at[pl.ds(nb, NODES_PER_TILE)])
    plsc.subcore_barrier()

    def _count(j, carry):
        pltpu.sync_copy(ones_v, deg_out_sh.at[src_v.at[j]], add=True)
        pltpu.sync_copy(ones_v, deg_in_sh.at[dst_v.at[j]], add=True)
        return carry
    lax.fori_loop(0, NCHUNK, _count, 0)
    plsc.subcore_barrier()

    pltpu.sync_copy(deg_out_sh.at[pl.ds(nb, NODES_PER_TILE)], dbuf)
    def _norm_out(j, carry):
        d = jnp.maximum(dbuf[pl.ds(j * 16, 16)], 1.0)
        nbuf[pl.ds(j * 16, 16)] = _rsqrt16(d)
        return carry
    lax.fori_loop(0, NODES_PER_TILE // 16, _norm_out, 0)
    pltpu.sync_copy(nbuf, deg_out_sh.at[pl.ds(nb, NODES_PER_TILE)])

    pltpu.sync_copy(deg_in_sh.at[pl.ds(nb, NODES_PER_TILE)], dbuf)
    lax.fori_loop(0, NODES_PER_TILE // 16, _norm_out, 0)
    pltpu.sync_copy(nbuf, deg_in_sh.at[pl.ds(nb, NODES_PER_TILE)])
    plsc.subcore_barrier()

    def _accum(j, carry):
        pltpu.sync_copy(deg_out_sh.at[src_v.at[j]], vals_v)
        pltpu.sync_copy(vals_v, s_sh.at[dst_v.at[j]], add=True)
        return carry
    lax.fori_loop(0, NCHUNK, _accum, 0)
    plsc.subcore_barrier()

    ob = cid * (NPAD // 2) + sid * OUT_PER_TILE
    pltpu.sync_copy(s_sh.at[pl.ds(ob, OUT_PER_TILE)], sbuf)
    pltpu.sync_copy(deg_in_sh.at[pl.ds(ob, OUT_PER_TILE)], ibuf)
    def _mul(j, carry):
        cbuf[pl.ds(j * 16, 16)] = sbuf[pl.ds(j * 16, 16)] * ibuf[pl.ds(j * 16, 16)]
        return carry
    lax.fori_loop(0, OUT_PER_TILE // 16, _mul, 0)
    pltpu.sync_copy(cbuf, out_hbm.at[pl.ds(ob, OUT_PER_TILE)])


@functools.cache
def _make_sc_compute_c():
    return functools.partial(
        pl.kernel,
        out_type=jax.ShapeDtypeStruct((NPAD,), jnp.float32),
        mesh=plsc.VectorSubcoreMesh(core_axis_name="c", subcore_axis_name="s"),
        scratch_types=[
            pltpu.VMEM((NCHUNK, CHUNK), jnp.int32),
            pltpu.VMEM((NCHUNK, CHUNK), jnp.int32),
            pltpu.VMEM((CHUNK,), jnp.float32),
            pltpu.VMEM((CHUNK,), jnp.float32),
            pltpu.VMEM((NODES_PER_TILE,), jnp.float32),
            pltpu.VMEM((NODES_PER_TILE,), jnp.float32),
            pltpu.VMEM((OUT_PER_TILE,), jnp.float32),
            pltpu.VMEM((OUT_PER_TILE,), jnp.float32),
            pltpu.VMEM((OUT_PER_TILE,), jnp.float32),
            pltpu.VMEM_SHARED((NPAD,), jnp.float32),
            pltpu.VMEM_SHARED((NPAD,), jnp.float32),
            pltpu.VMEM_SHARED((NPAD,), jnp.float32),
        ],
    )(_sc_body)


BN = 1000


def _tc_body(c_ref, x_ref, wr_ref, br_ref, wz_ref, bz_ref, wh_ref, bh_ref,
             wg_ref, bg_ref, o_ref):
    f32 = jnp.float32
    xm = x_ref[...]
    bgv = bg_ref[...][None, :]
    xr = jnp.dot(xm, wr_ref[...], preferred_element_type=f32) + br_ref[...][None, :]
    xz = jnp.dot(xm, wz_ref[...], preferred_element_type=f32) + bz_ref[...][None, :]
    xh = jnp.dot(xm, wh_ref[...], preferred_element_type=f32) + bh_ref[...][None, :]
    r0 = jax.nn.sigmoid(xr + bgv)
    z0 = jax.nn.sigmoid(xz + bgv)
    h1 = z0 * jnp.tanh(xh + r0 * bgv)
    g = jnp.dot(h1, wg_ref[...], preferred_element_type=f32)
    cb = c_ref[...]
    for b in range(B):
        hc = g[b:b + 1, :] * cb + bgv
        rb = jax.nn.sigmoid(xr[b:b + 1, :] + hc)
        zb = jax.nn.sigmoid(xz[b:b + 1, :] + hc)
        ht = jnp.tanh(xh[b:b + 1, :] + rb * hc)
        h2 = (1.0 - zb) * h1[b:b + 1, :] + zb * ht
        o_ref[b, 0] = jnp.broadcast_to(h1[b:b + 1, :], (BN, H))
        o_ref[b, 1] = h2


def _tc_gru(c2, x, Wr, br, Wz, bz, Wh, bh, Wg, bg):
    full2 = lambda i: (0, 0)
    full1 = lambda i: (0,)
    return pl.pallas_call(
        _tc_body,
        grid=(N // BN,),
        in_specs=[
            pl.BlockSpec((BN, 1), lambda i: (i, 0)),
            pl.BlockSpec((B, D_IN), full2),
            pl.BlockSpec((D_IN, H), full2),
            pl.BlockSpec((H,), full1),
            pl.BlockSpec((D_IN, H), full2),
            pl.BlockSpec((H,), full1),
            pl.BlockSpec((D_IN, H), full2),
            pl.BlockSpec((H,), full1),
            pl.BlockSpec((H, H), full2),
            pl.BlockSpec((H,), full1),
        ],
        out_specs=pl.BlockSpec((B, 2, BN, H), lambda i: (0, 0, i, 0)),
        out_shape=jax.ShapeDtypeStruct((B, 2, N, H), jnp.float32),
    )(c2, x, Wr, br, Wz, bz, Wh, bh, Wg, bg)


def kernel(x, edge_index, Wr, br, Wz, bz, Wh, bh, Wg, bg):
    src = edge_index[0]
    dst = edge_index[1]
    pad = jnp.full((NTILES * EPT - E,), DUMMY, jnp.int32)
    src_r = jnp.concatenate([src, pad]).reshape(NTILES, NCHUNK, CHUNK)
    dst_r = jnp.concatenate([dst, pad]).reshape(NTILES, NCHUNK, CHUNK)
    c = _make_sc_compute_c()(src_r, dst_r)
    c2 = c[:N].reshape(N, 1)
    out4 = _tc_gru(c2, x, Wr, br, Wz, bz, Wh, bh, Wg, bg)
    return out4.reshape(B, -1)
